# Initial kernel scaffold; baseline (speedup 1.0000x reference)
import jax
import jax.numpy as jnp
from jax.experimental import pallas as pl


def kernel(x, edge_index, W1, b1, W2, b2, W3, b3, Wc1, bc1, Wc2, bc2):
    raise NotImplementedError("write your pallas kernel here")


# speedup vs baseline: 12.9101x; 12.9101x over previous
import functools

import jax
import jax.numpy as jnp
from jax import lax
from jax.experimental import pallas as pl
from jax.experimental.pallas import tpu as pltpu
from jax.experimental.pallas import tpu_sc as plsc

N = 10000
E = 320000
H = 128
C_OUT = 12

NC = 2
NS = 16
NW = NC * NS
N_PAD = 10240
RPT = N_PAD // NS
CW = 80
EPW = E // NW
NCH = EPW // CW
BR = 2000


def _sc_scatter_body(
    src_hbm, dst_hbm, y_hbm, out_hbm, sidx, didx, sbuf, dbuf, rows, acc, sem
):
    cid = lax.axis_index("c")
    sid = lax.axis_index("s")
    wid = sid * NC + cid
    base = sid * RPT

    zeros16 = jnp.zeros((16,), jnp.float32)

    def zfill(i, _):
        for k in range(H // 16):
            rows[i, pl.ds(k * 16, 16)] = zeros16
        return 0

    lax.fori_loop(0, CW, zfill, 0)
    pltpu.sync_copy(src_hbm.at[wid], sidx)
    pltpu.sync_copy(dst_hbm.at[wid], didx)
    for k in range(RPT // CW):
        pltpu.sync_copy(rows, acc.at[pl.ds(base + k * CW, CW)])
    plsc.subcore_barrier()

    def body(j, _):
        for k in range(CW // 16):
            sbuf[pl.ds(k * 16, 16)] = sidx[j, pl.ds(k * 16, 16)]
            dbuf[pl.ds(k * 16, 16)] = didx[j, pl.ds(k * 16, 16)]
        pltpu.async_copy(y_hbm.at[sbuf], rows, sem).wait()
        pltpu.sync_copy(rows, acc.at[dbuf], add=True)
        return 0

    lax.fori_loop(0, NCH, body, 0)
    plsc.subcore_barrier()
    pltpu.sync_copy(
        acc.at[pl.ds(base, RPT)], out_hbm.at[pl.ds(cid * N_PAD + base, RPT)]
    )


@functools.cache
def _get_sc_scatter():
    mesh = plsc.VectorSubcoreMesh(
        core_axis_name="c", subcore_axis_name="s", num_cores=NC, num_subcores=NS
    )
    return pl.kernel(
        _sc_scatter_body,
        out_type=jax.ShapeDtypeStruct((NC * N_PAD, H), jnp.float32),
        mesh=mesh,
        scratch_types=[
            pltpu.VMEM((NCH, CW), jnp.int32),
            pltpu.VMEM((NCH, CW), jnp.int32),
            pltpu.VMEM((CW,), jnp.int32),
            pltpu.VMEM((CW,), jnp.int32),
            pltpu.VMEM((CW, H), jnp.float32),
            pltpu.VMEM_SHARED((N_PAD, H), jnp.float32),
            pltpu.SemaphoreType.DMA,
        ],
    )


def _tc_first_body(x_ref, w_ref, d0_ref, d1_ref, y_ref, dinv_ref):
    deg = d0_ref[...] + d1_ref[...] + 1.0
    dinv = lax.rsqrt(deg)
    xw = jnp.dot(x_ref[...], w_ref[...], preferred_element_type=jnp.float32)
    y_ref[...] = xw * dinv
    dinv_ref[...] = dinv


_tc_first = pl.pallas_call(
    _tc_first_body,
    grid=(N // BR,),
    in_specs=[
        pl.BlockSpec((BR, H), lambda i: (i, 0)),
        pl.BlockSpec((H, H), lambda i: (0, 0)),
        pl.BlockSpec((BR, 1), lambda i: (i, 0)),
        pl.BlockSpec((BR, 1), lambda i: (i, 0)),
    ],
    out_specs=[
        pl.BlockSpec((BR, H), lambda i: (i, 0)),
        pl.BlockSpec((BR, 1), lambda i: (i, 0)),
    ],
    out_shape=[
        jax.ShapeDtypeStruct((N, H), jnp.float32),
        jax.ShapeDtypeStruct((N, 1), jnp.float32),
    ],
)


def _tc_mid_body(a0_ref, a1_ref, y_ref, dinv_ref, b_ref, w_ref, out_ref):
    dinv = dinv_ref[...]
    h = jnp.tanh(dinv * (a0_ref[...] + a1_ref[...] + y_ref[...]) + b_ref[...])
    out_ref[...] = dinv * jnp.dot(
        h, w_ref[...], preferred_element_type=jnp.float32
    )


_tc_mid = pl.pallas_call(
    _tc_mid_body,
    grid=(N // BR,),
    in_specs=[
        pl.BlockSpec((BR, H), lambda i: (i, 0)),
        pl.BlockSpec((BR, H), lambda i: (i, 0)),
        pl.BlockSpec((BR, H), lambda i: (i, 0)),
        pl.BlockSpec((BR, 1), lambda i: (i, 0)),
        pl.BlockSpec((1, H), lambda i: (0, 0)),
        pl.BlockSpec((H, H), lambda i: (0, 0)),
    ],
    out_specs=pl.BlockSpec((BR, H), lambda i: (i, 0)),
    out_shape=jax.ShapeDtypeStruct((N, H), jnp.float32),
)


def _tc_final_body(
    a0_ref, a1_ref, y_ref, dinv_ref, b_ref, wc1_ref, bc1_ref, wc2_ref, bc2_ref, out_ref
):
    dinv = dinv_ref[...]
    h = jnp.tanh(dinv * (a0_ref[...] + a1_ref[...] + y_ref[...]) + b_ref[...])
    o = jnp.dot(h, wc1_ref[...], preferred_element_type=jnp.float32) + bc1_ref[...]
    out_ref[...] = (
        jnp.dot(o, wc2_ref[...], preferred_element_type=jnp.float32) + bc2_ref[...]
    )


_tc_final = pl.pallas_call(
    _tc_final_body,
    grid=(N // BR,),
    in_specs=[
        pl.BlockSpec((BR, H), lambda i: (i, 0)),
        pl.BlockSpec((BR, H), lambda i: (i, 0)),
        pl.BlockSpec((BR, H), lambda i: (i, 0)),
        pl.BlockSpec((BR, 1), lambda i: (i, 0)),
        pl.BlockSpec((1, H), lambda i: (0, 0)),
        pl.BlockSpec((H, 64), lambda i: (0, 0)),
        pl.BlockSpec((1, 64), lambda i: (0, 0)),
        pl.BlockSpec((64, C_OUT), lambda i: (0, 0)),
        pl.BlockSpec((1, C_OUT), lambda i: (0, 0)),
    ],
    out_specs=pl.BlockSpec((BR, C_OUT), lambda i: (i, 0)),
    out_shape=jax.ShapeDtypeStruct((N, C_OUT), jnp.float32),
)


def kernel(x, edge_index, W1, b1, W2, b2, W3, b3, Wc1, bc1, Wc2, bc2):
    src2d = edge_index[0].reshape(NW, NCH, CW)
    dst2d = edge_index[1].reshape(NW, NCH, CW)

    sc_scatter = _get_sc_scatter()
    degp = sc_scatter(dst2d, dst2d, jnp.ones((N, H), jnp.float32))
    d0 = degp[:N, :1]
    d1 = degp[N_PAD : N_PAD + N, :1]

    y1, dinv = _tc_first(x, W1, d0, d1)

    b1r = b1.reshape(1, H)
    b2r = b2.reshape(1, H)
    b3r = b3.reshape(1, H)
    bc1r = bc1.reshape(1, 64)
    bc2r = bc2.reshape(1, C_OUT)

    a = sc_scatter(src2d, dst2d, y1)
    y2 = _tc_mid(a[:N], a[N_PAD : N_PAD + N], y1, dinv, b1r, W2)
    a = sc_scatter(src2d, dst2d, y2)
    y3 = _tc_mid(a[:N], a[N_PAD : N_PAD + N], y2, dinv, b2r, W3)
    a = sc_scatter(src2d, dst2d, y3)
    out = _tc_final(
        a[:N], a[N_PAD : N_PAD + N], y3, dinv, b3r, Wc1, bc1r, Wc2, bc2r
    )
    return out


# speedup vs baseline: 17.0258x; 1.3188x over previous
import functools

import jax
import jax.numpy as jnp
from jax import lax
from jax.experimental import pallas as pl
from jax.experimental.pallas import tpu as pltpu
from jax.experimental.pallas import tpu_sc as plsc

N = 10000
E = 320000
H = 128
C_OUT = 12

NC = 2
NS = 16
NW = NC * NS
N_PAD = 10240
RPT = N_PAD // NS
CW = 80
EPW = E // NW
NCH = EPW // CW
BR = 2000


def _sc_scatter_body(
    src_hbm, dst_hbm, y_hbm, out_hbm,
    sbuf0, sbuf1, dbuf0, dbuf1, rows0, rows1, acc,
    isem0, isem1, gsem0, gsem1, ssem0, ssem1,
):
    cid = lax.axis_index("c")
    sid = lax.axis_index("s")
    wid = sid * NC + cid
    base = sid * RPT
    ebase = wid * EPW

    zeros16 = jnp.zeros((16,), jnp.float32)

    def zfill(i, _):
        for k in range(H // 16):
            rows0[i, pl.ds(k * 16, 16)] = zeros16
        return 0

    lax.fori_loop(0, CW, zfill, 0)
    for k in range(RPT // CW):
        pltpu.sync_copy(rows0, acc.at[pl.ds(base + k * CW, CW)])
    plsc.subcore_barrier()

    def fetch(j, sb, db, sem):
        pltpu.async_copy(src_hbm.at[pl.ds(ebase + j * CW, CW)], sb, sem)
        pltpu.async_copy(dst_hbm.at[pl.ds(ebase + j * CW, CW)], db, sem)

    def wait_fetch(j, sb, db, sem):
        pltpu.make_async_copy(src_hbm.at[pl.ds(ebase + j * CW, CW)], sb, sem).wait()
        pltpu.make_async_copy(dst_hbm.at[pl.ds(ebase + j * CW, CW)], db, sem).wait()

    def gather(sb, rows, sem):
        pltpu.async_copy(y_hbm.at[sb], rows, sem)

    def wait_gather(sb, rows, sem):
        pltpu.make_async_copy(y_hbm.at[sb], rows, sem).wait()

    def scatter(db, rows, sem):
        pltpu.async_copy(rows, acc.at[db], sem, add=True)

    def wait_scatter(db, rows, sem):
        pltpu.make_async_copy(rows, acc.at[db], sem).wait()

    fetch(0, sbuf0, dbuf0, isem0)
    fetch(1, sbuf1, dbuf1, isem1)
    wait_fetch(0, sbuf0, dbuf0, isem0)
    gather(sbuf0, rows0, gsem0)
    wait_fetch(1, sbuf1, dbuf1, isem1)
    gather(sbuf1, rows1, gsem1)
    wait_gather(sbuf0, rows0, gsem0)
    scatter(dbuf0, rows0, ssem0)
    wait_gather(sbuf1, rows1, gsem1)
    scatter(dbuf1, rows1, ssem1)

    def body(k, _):
        j0 = 2 * k
        j1 = 2 * k + 1
        wait_scatter(dbuf0, rows0, ssem0)
        fetch(j0, sbuf0, dbuf0, isem0)
        wait_scatter(dbuf1, rows1, ssem1)
        fetch(j1, sbuf1, dbuf1, isem1)
        wait_fetch(j0, sbuf0, dbuf0, isem0)
        gather(sbuf0, rows0, gsem0)
        wait_fetch(j1, sbuf1, dbuf1, isem1)
        gather(sbuf1, rows1, gsem1)
        wait_gather(sbuf0, rows0, gsem0)
        scatter(dbuf0, rows0, ssem0)
        wait_gather(sbuf1, rows1, gsem1)
        scatter(dbuf1, rows1, ssem1)
        return 0

    lax.fori_loop(1, (NCH - 1) // 2, body, 0)

    wait_scatter(dbuf0, rows0, ssem0)
    fetch(NCH - 1, sbuf0, dbuf0, isem0)
    wait_fetch(NCH - 1, sbuf0, dbuf0, isem0)
    gather(sbuf0, rows0, gsem0)
    wait_gather(sbuf0, rows0, gsem0)
    scatter(dbuf0, rows0, ssem0)
    wait_scatter(dbuf1, rows1, ssem1)
    wait_scatter(dbuf0, rows0, ssem0)

    plsc.subcore_barrier()
    pltpu.sync_copy(
        acc.at[pl.ds(base, RPT)], out_hbm.at[pl.ds(cid * N_PAD + base, RPT)]
    )


@functools.cache
def _get_sc_scatter():
    mesh = plsc.VectorSubcoreMesh(
        core_axis_name="c", subcore_axis_name="s", num_cores=NC, num_subcores=NS
    )
    return pl.kernel(
        _sc_scatter_body,
        out_type=jax.ShapeDtypeStruct((NC * N_PAD, H), jnp.float32),
        mesh=mesh,
        scratch_types=[
            pltpu.VMEM((CW,), jnp.int32),
            pltpu.VMEM((CW,), jnp.int32),
            pltpu.VMEM((CW,), jnp.int32),
            pltpu.VMEM((CW,), jnp.int32),
            pltpu.VMEM((CW, H), jnp.float32),
            pltpu.VMEM((CW, H), jnp.float32),
            pltpu.VMEM_SHARED((N_PAD, H), jnp.float32),
            pltpu.SemaphoreType.DMA,
            pltpu.SemaphoreType.DMA,
            pltpu.SemaphoreType.DMA,
            pltpu.SemaphoreType.DMA,
            pltpu.SemaphoreType.DMA,
            pltpu.SemaphoreType.DMA,
        ],
    )


def _sc_deg_body(dst_hbm, out_hbm, dbuf0, dbuf1, buf, acc, isem0, isem1, ssem0, ssem1):
    cid = lax.axis_index("c")
    sid = lax.axis_index("s")
    wid = sid * NC + cid
    base = sid * RPT
    ebase = wid * EPW

    zeros16 = jnp.zeros((16,), jnp.float32)
    ones16 = jnp.ones((16,), jnp.float32)

    def zfill(i, _):
        for k in range(H // 16):
            buf[i, pl.ds(k * 16, 16)] = zeros16
        return 0

    lax.fori_loop(0, CW, zfill, 0)
    for k in range(RPT // CW):
        pltpu.sync_copy(buf, acc.at[pl.ds(base + k * CW, CW)])
    plsc.subcore_barrier()

    def ofill(i, _):
        for k in range(H // 16):
            buf[i, pl.ds(k * 16, 16)] = ones16
        return 0

    lax.fori_loop(0, CW, ofill, 0)

    def fetch(j, db, sem):
        pltpu.async_copy(dst_hbm.at[pl.ds(ebase + j * CW, CW)], db, sem)

    def wait_fetch(j, db, sem):
        pltpu.make_async_copy(dst_hbm.at[pl.ds(ebase + j * CW, CW)], db, sem).wait()

    def scatter(db, sem):
        pltpu.async_copy(buf, acc.at[db], sem, add=True)

    def wait_scatter(db, sem):
        pltpu.make_async_copy(buf, acc.at[db], sem).wait()

    fetch(0, dbuf0, isem0)
    fetch(1, dbuf1, isem1)
    wait_fetch(0, dbuf0, isem0)
    scatter(dbuf0, ssem0)
    wait_fetch(1, dbuf1, isem1)
    scatter(dbuf1, ssem1)

    def body(k, _):
        j0 = 2 * k
        j1 = 2 * k + 1
        wait_scatter(dbuf0, ssem0)
        fetch(j0, dbuf0, isem0)
        wait_scatter(dbuf1, ssem1)
        fetch(j1, dbuf1, isem1)
        wait_fetch(j0, dbuf0, isem0)
        scatter(dbuf0, ssem0)
        wait_fetch(j1, dbuf1, isem1)
        scatter(dbuf1, ssem1)
        return 0

    lax.fori_loop(1, (NCH - 1) // 2, body, 0)

    wait_scatter(dbuf0, ssem0)
    fetch(NCH - 1, dbuf0, isem0)
    wait_fetch(NCH - 1, dbuf0, isem0)
    scatter(dbuf0, ssem0)
    wait_scatter(dbuf1, ssem1)
    wait_scatter(dbuf0, ssem0)

    plsc.subcore_barrier()
    pltpu.sync_copy(
        acc.at[pl.ds(base, RPT)], out_hbm.at[pl.ds(cid * N_PAD + base, RPT)]
    )


@functools.cache
def _get_sc_deg():
    mesh = plsc.VectorSubcoreMesh(
        core_axis_name="c", subcore_axis_name="s", num_cores=NC, num_subcores=NS
    )
    return pl.kernel(
        _sc_deg_body,
        out_type=jax.ShapeDtypeStruct((NC * N_PAD, H), jnp.float32),
        mesh=mesh,
        scratch_types=[
            pltpu.VMEM((CW,), jnp.int32),
            pltpu.VMEM((CW,), jnp.int32),
            pltpu.VMEM((CW, H), jnp.float32),
            pltpu.VMEM_SHARED((N_PAD, H), jnp.float32),
            pltpu.SemaphoreType.DMA,
            pltpu.SemaphoreType.DMA,
            pltpu.SemaphoreType.DMA,
            pltpu.SemaphoreType.DMA,
        ],
    )


def _tc_first_body(x_ref, w_ref, d0_ref, d1_ref, y_ref, dinv_ref):
    deg = d0_ref[...] + d1_ref[...] + 1.0
    dinv = lax.rsqrt(deg)
    xw = jnp.dot(x_ref[...], w_ref[...], preferred_element_type=jnp.float32)
    y_ref[...] = xw * dinv
    dinv_ref[...] = dinv


_tc_first = pl.pallas_call(
    _tc_first_body,
    grid=(N // BR,),
    in_specs=[
        pl.BlockSpec((BR, H), lambda i: (i, 0)),
        pl.BlockSpec((H, H), lambda i: (0, 0)),
        pl.BlockSpec((BR, 1), lambda i: (i, 0)),
        pl.BlockSpec((BR, 1), lambda i: (i, 0)),
    ],
    out_specs=[
        pl.BlockSpec((BR, H), lambda i: (i, 0)),
        pl.BlockSpec((BR, 1), lambda i: (i, 0)),
    ],
    out_shape=[
        jax.ShapeDtypeStruct((N, H), jnp.float32),
        jax.ShapeDtypeStruct((N, 1), jnp.float32),
    ],
)


def _tc_mid_body(a0_ref, a1_ref, y_ref, dinv_ref, b_ref, w_ref, out_ref):
    dinv = dinv_ref[...]
    h = jnp.tanh(dinv * (a0_ref[...] + a1_ref[...] + y_ref[...]) + b_ref[...])
    out_ref[...] = dinv * jnp.dot(
        h, w_ref[...], preferred_element_type=jnp.float32
    )


_tc_mid = pl.pallas_call(
    _tc_mid_body,
    grid=(N // BR,),
    in_specs=[
        pl.BlockSpec((BR, H), lambda i: (i, 0)),
        pl.BlockSpec((BR, H), lambda i: (i, 0)),
        pl.BlockSpec((BR, H), lambda i: (i, 0)),
        pl.BlockSpec((BR, 1), lambda i: (i, 0)),
        pl.BlockSpec((1, H), lambda i: (0, 0)),
        pl.BlockSpec((H, H), lambda i: (0, 0)),
    ],
    out_specs=pl.BlockSpec((BR, H), lambda i: (i, 0)),
    out_shape=jax.ShapeDtypeStruct((N, H), jnp.float32),
)


def _tc_final_body(
    a0_ref, a1_ref, y_ref, dinv_ref, b_ref, wc1_ref, bc1_ref, wc2_ref, bc2_ref, out_ref
):
    dinv = dinv_ref[...]
    h = jnp.tanh(dinv * (a0_ref[...] + a1_ref[...] + y_ref[...]) + b_ref[...])
    o = jnp.dot(h, wc1_ref[...], preferred_element_type=jnp.float32) + bc1_ref[...]
    out_ref[...] = (
        jnp.dot(o, wc2_ref[...], preferred_element_type=jnp.float32) + bc2_ref[...]
    )


_tc_final = pl.pallas_call(
    _tc_final_body,
    grid=(N // BR,),
    in_specs=[
        pl.BlockSpec((BR, H), lambda i: (i, 0)),
        pl.BlockSpec((BR, H), lambda i: (i, 0)),
        pl.BlockSpec((BR, H), lambda i: (i, 0)),
        pl.BlockSpec((BR, 1), lambda i: (i, 0)),
        pl.BlockSpec((1, H), lambda i: (0, 0)),
        pl.BlockSpec((H, 64), lambda i: (0, 0)),
        pl.BlockSpec((1, 64), lambda i: (0, 0)),
        pl.BlockSpec((64, C_OUT), lambda i: (0, 0)),
        pl.BlockSpec((1, C_OUT), lambda i: (0, 0)),
    ],
    out_specs=pl.BlockSpec((BR, C_OUT), lambda i: (i, 0)),
    out_shape=jax.ShapeDtypeStruct((N, C_OUT), jnp.float32),
)


def kernel(x, edge_index, W1, b1, W2, b2, W3, b3, Wc1, bc1, Wc2, bc2):
    src1 = edge_index[0]
    dst1 = edge_index[1]

    sc_scatter = _get_sc_scatter()
    degp = _get_sc_deg()(dst1)
    d0 = degp[:N, :1]
    d1 = degp[N_PAD : N_PAD + N, :1]

    y1, dinv = _tc_first(x, W1, d0, d1)

    b1r = b1.reshape(1, H)
    b2r = b2.reshape(1, H)
    b3r = b3.reshape(1, H)
    bc1r = bc1.reshape(1, 64)
    bc2r = bc2.reshape(1, C_OUT)

    a = sc_scatter(src1, dst1, y1)
    y2 = _tc_mid(a[:N], a[N_PAD : N_PAD + N], y1, dinv, b1r, W2)
    a = sc_scatter(src1, dst1, y2)
    y3 = _tc_mid(a[:N], a[N_PAD : N_PAD + N], y2, dinv, b2r, W3)
    a = sc_scatter(src1, dst1, y3)
    out = _tc_final(
        a[:N], a[N_PAD : N_PAD + N], y3, dinv, b3r, Wc1, bc1r, Wc2, bc2r
    )
    return out


# speedup vs baseline: 21.7148x; 1.2754x over previous
import functools

import jax
import jax.numpy as jnp
from jax import lax
from jax.experimental import pallas as pl
from jax.experimental.pallas import tpu as pltpu
from jax.experimental.pallas import tpu_sc as plsc

N = 10000
E = 320000
H = 128
C_OUT = 12

NC = 2
NS = 16
NW = NC * NS
N_PAD = 10240
RPT = N_PAD // NS
CW = 80
EPW = E // NW
NCH = EPW // CW
BR = 2000


def _sc_scatter_body(
    src_hbm, dst_hbm, y_hbm, out_hbm,
    sbuf0, sbuf1, sbuf2, sbuf3, dbuf0, dbuf1, dbuf2, dbuf3,
    rows0, rows1, rows2, rows3, acc,
    isem0, isem1, isem2, isem3, gsem0, gsem1, gsem2, gsem3,
    ssem0, ssem1, ssem2, ssem3,
):
    cid = lax.axis_index("c")
    sid = lax.axis_index("s")
    wid = sid * NC + cid
    base = sid * RPT
    ebase = wid * EPW

    slots = (
        (sbuf0, dbuf0, rows0, isem0, gsem0, ssem0),
        (sbuf1, dbuf1, rows1, isem1, gsem1, ssem1),
        (sbuf2, dbuf2, rows2, isem2, gsem2, ssem2),
        (sbuf3, dbuf3, rows3, isem3, gsem3, ssem3),
    )
    NSL = len(slots)

    zeros16 = jnp.zeros((16,), jnp.float32)

    def zfill(i, _):
        for k in range(H // 16):
            rows0[i, pl.ds(k * 16, 16)] = zeros16
        return 0

    lax.fori_loop(0, CW, zfill, 0)
    for k in range(RPT // CW):
        pltpu.sync_copy(rows0, acc.at[pl.ds(base + k * CW, CW)])
    plsc.subcore_barrier()

    def fetch(j, sl):
        sb, db, _, isem, _, _ = sl
        pltpu.async_copy(src_hbm.at[pl.ds(ebase + j * CW, CW)], sb, isem)
        pltpu.async_copy(dst_hbm.at[pl.ds(ebase + j * CW, CW)], db, isem)

    def wait_fetch(j, sl):
        sb, db, _, isem, _, _ = sl
        pltpu.make_async_copy(src_hbm.at[pl.ds(ebase + j * CW, CW)], sb, isem).wait()
        pltpu.make_async_copy(dst_hbm.at[pl.ds(ebase + j * CW, CW)], db, isem).wait()

    def gather(sl):
        sb, _, rows, _, gsem, _ = sl
        pltpu.async_copy(y_hbm.at[sb], rows, gsem)

    def wait_gather(sl):
        sb, _, rows, _, gsem, _ = sl
        pltpu.make_async_copy(y_hbm.at[sb], rows, gsem).wait()

    def scatter(sl):
        _, db, rows, _, _, ssem = sl
        pltpu.async_copy(rows, acc.at[db], ssem, add=True)

    def wait_scatter(sl):
        _, db, rows, _, _, ssem = sl
        pltpu.make_async_copy(rows, acc.at[db], ssem).wait()

    for sidx in range(NSL):
        fetch(sidx, slots[sidx])
    for sidx in range(NSL):
        wait_fetch(sidx, slots[sidx])
        gather(slots[sidx])
    for sidx in range(NSL):
        wait_gather(slots[sidx])
        scatter(slots[sidx])

    def body(k, _):
        j = NSL * k
        for sidx in range(NSL):
            wait_scatter(slots[sidx])
            fetch(j + sidx, slots[sidx])
        for sidx in range(NSL):
            wait_fetch(j + sidx, slots[sidx])
            gather(slots[sidx])
        for sidx in range(NSL):
            wait_gather(slots[sidx])
            scatter(slots[sidx])
        return 0

    lax.fori_loop(1, NCH // NSL, body, 0)

    wait_scatter(slots[0])
    fetch(NCH - 1, slots[0])
    wait_fetch(NCH - 1, slots[0])
    gather(slots[0])
    wait_gather(slots[0])
    scatter(slots[0])
    for sidx in range(1, NSL):
        wait_scatter(slots[sidx])
    wait_scatter(slots[0])

    plsc.subcore_barrier()
    pltpu.sync_copy(
        acc.at[pl.ds(base, RPT)], out_hbm.at[cid, pl.ds(base, RPT)]
    )


@functools.cache
def _get_sc_scatter():
    mesh = plsc.VectorSubcoreMesh(
        core_axis_name="c", subcore_axis_name="s", num_cores=NC, num_subcores=NS
    )
    return pl.kernel(
        _sc_scatter_body,
        out_type=jax.ShapeDtypeStruct((NC, N_PAD, H), jnp.float32),
        mesh=mesh,
        scratch_types=(
            [pltpu.VMEM((CW,), jnp.int32)] * 8
            + [pltpu.VMEM((CW, H), jnp.float32)] * 4
            + [pltpu.VMEM_SHARED((N_PAD, H), jnp.float32)]
            + [pltpu.SemaphoreType.DMA] * 12
        ),
    )


def _sc_deg_body(dst_hbm, out_hbm, dbuf0, dbuf1, buf, acc, isem0, isem1, ssem0, ssem1):
    cid = lax.axis_index("c")
    sid = lax.axis_index("s")
    wid = sid * NC + cid
    base = sid * RPT
    ebase = wid * EPW

    zeros16 = jnp.zeros((16,), jnp.float32)
    ones16 = jnp.ones((16,), jnp.float32)

    def zfill(i, _):
        for k in range(H // 16):
            buf[i, pl.ds(k * 16, 16)] = zeros16
        return 0

    lax.fori_loop(0, CW, zfill, 0)
    for k in range(RPT // CW):
        pltpu.sync_copy(buf, acc.at[pl.ds(base + k * CW, CW)])
    plsc.subcore_barrier()

    def ofill(i, _):
        for k in range(H // 16):
            buf[i, pl.ds(k * 16, 16)] = ones16
        return 0

    lax.fori_loop(0, CW, ofill, 0)

    def fetch(j, db, sem):
        pltpu.async_copy(dst_hbm.at[pl.ds(ebase + j * CW, CW)], db, sem)

    def wait_fetch(j, db, sem):
        pltpu.make_async_copy(dst_hbm.at[pl.ds(ebase + j * CW, CW)], db, sem).wait()

    def scatter(db, sem):
        pltpu.async_copy(buf, acc.at[db], sem, add=True)

    def wait_scatter(db, sem):
        pltpu.make_async_copy(buf, acc.at[db], sem).wait()

    fetch(0, dbuf0, isem0)
    fetch(1, dbuf1, isem1)
    wait_fetch(0, dbuf0, isem0)
    scatter(dbuf0, ssem0)
    wait_fetch(1, dbuf1, isem1)
    scatter(dbuf1, ssem1)

    def body(k, _):
        j0 = 2 * k
        j1 = 2 * k + 1
        wait_scatter(dbuf0, ssem0)
        fetch(j0, dbuf0, isem0)
        wait_scatter(dbuf1, ssem1)
        fetch(j1, dbuf1, isem1)
        wait_fetch(j0, dbuf0, isem0)
        scatter(dbuf0, ssem0)
        wait_fetch(j1, dbuf1, isem1)
        scatter(dbuf1, ssem1)
        return 0

    lax.fori_loop(1, (NCH - 1) // 2, body, 0)

    wait_scatter(dbuf0, ssem0)
    fetch(NCH - 1, dbuf0, isem0)
    wait_fetch(NCH - 1, dbuf0, isem0)
    scatter(dbuf0, ssem0)
    wait_scatter(dbuf1, ssem1)
    wait_scatter(dbuf0, ssem0)

    plsc.subcore_barrier()
    pltpu.sync_copy(
        acc.at[pl.ds(base, RPT)], out_hbm.at[cid, pl.ds(base, RPT)]
    )


@functools.cache
def _get_sc_deg():
    mesh = plsc.VectorSubcoreMesh(
        core_axis_name="c", subcore_axis_name="s", num_cores=NC, num_subcores=NS
    )
    return pl.kernel(
        _sc_deg_body,
        out_type=jax.ShapeDtypeStruct((NC, N_PAD, H), jnp.float32),
        mesh=mesh,
        scratch_types=[
            pltpu.VMEM((CW,), jnp.int32),
            pltpu.VMEM((CW,), jnp.int32),
            pltpu.VMEM((CW, H), jnp.float32),
            pltpu.VMEM_SHARED((N_PAD, H), jnp.float32),
            pltpu.SemaphoreType.DMA,
            pltpu.SemaphoreType.DMA,
            pltpu.SemaphoreType.DMA,
            pltpu.SemaphoreType.DMA,
        ],
    )


def _tc_first_body(x_ref, w_ref, d0_ref, d1_ref, y_ref, dinv_ref):
    deg = d0_ref[0][:, :1] + d1_ref[0][:, :1] + 1.0
    dinv = lax.rsqrt(deg)
    xw = jnp.dot(x_ref[...], w_ref[...], preferred_element_type=jnp.float32)
    y_ref[...] = xw * dinv
    dinv_ref[...] = dinv


_tc_first = pl.pallas_call(
    _tc_first_body,
    grid=(N // BR,),
    in_specs=[
        pl.BlockSpec((BR, H), lambda i: (i, 0)),
        pl.BlockSpec((H, H), lambda i: (0, 0)),
        pl.BlockSpec((1, BR, H), lambda i: (0, i, 0)),
        pl.BlockSpec((1, BR, H), lambda i: (1, i, 0)),
    ],
    out_specs=[
        pl.BlockSpec((BR, H), lambda i: (i, 0)),
        pl.BlockSpec((BR, 1), lambda i: (i, 0)),
    ],
    out_shape=[
        jax.ShapeDtypeStruct((N, H), jnp.float32),
        jax.ShapeDtypeStruct((N, 1), jnp.float32),
    ],
)


def _tc_mid_body(a0_ref, a1_ref, y_ref, dinv_ref, b_ref, w_ref, out_ref):
    dinv = dinv_ref[...]
    h = jnp.tanh(dinv * (a0_ref[0] + a1_ref[0] + y_ref[...]) + b_ref[...])
    out_ref[...] = dinv * jnp.dot(
        h, w_ref[...], preferred_element_type=jnp.float32
    )


_tc_mid = pl.pallas_call(
    _tc_mid_body,
    grid=(N // BR,),
    in_specs=[
        pl.BlockSpec((1, BR, H), lambda i: (0, i, 0)),
        pl.BlockSpec((1, BR, H), lambda i: (1, i, 0)),
        pl.BlockSpec((BR, H), lambda i: (i, 0)),
        pl.BlockSpec((BR, 1), lambda i: (i, 0)),
        pl.BlockSpec((1, H), lambda i: (0, 0)),
        pl.BlockSpec((H, H), lambda i: (0, 0)),
    ],
    out_specs=pl.BlockSpec((BR, H), lambda i: (i, 0)),
    out_shape=jax.ShapeDtypeStruct((N, H), jnp.float32),
)


def _tc_final_body(
    a0_ref, a1_ref, y_ref, dinv_ref, b_ref, wc1_ref, bc1_ref, wc2_ref, bc2_ref, out_ref
):
    dinv = dinv_ref[...]
    h = jnp.tanh(dinv * (a0_ref[0] + a1_ref[0] + y_ref[...]) + b_ref[...])
    o = jnp.dot(h, wc1_ref[...], preferred_element_type=jnp.float32) + bc1_ref[...]
    out_ref[...] = (
        jnp.dot(o, wc2_ref[...], preferred_element_type=jnp.float32) + bc2_ref[...]
    )


_tc_final = pl.pallas_call(
    _tc_final_body,
    grid=(N // BR,),
    in_specs=[
        pl.BlockSpec((1, BR, H), lambda i: (0, i, 0)),
        pl.BlockSpec((1, BR, H), lambda i: (1, i, 0)),
        pl.BlockSpec((BR, H), lambda i: (i, 0)),
        pl.BlockSpec((BR, 1), lambda i: (i, 0)),
        pl.BlockSpec((1, H), lambda i: (0, 0)),
        pl.BlockSpec((H, 64), lambda i: (0, 0)),
        pl.BlockSpec((1, 64), lambda i: (0, 0)),
        pl.BlockSpec((64, C_OUT), lambda i: (0, 0)),
        pl.BlockSpec((1, C_OUT), lambda i: (0, 0)),
    ],
    out_specs=pl.BlockSpec((BR, C_OUT), lambda i: (i, 0)),
    out_shape=jax.ShapeDtypeStruct((N, C_OUT), jnp.float32),
)


def kernel(x, edge_index, W1, b1, W2, b2, W3, b3, Wc1, bc1, Wc2, bc2):
    src1 = edge_index[0]
    dst1 = edge_index[1]

    sc_scatter = _get_sc_scatter()
    degp = _get_sc_deg()(dst1)

    y1, dinv = _tc_first(x, W1, degp, degp)

    b1r = b1.reshape(1, H)
    b2r = b2.reshape(1, H)
    b3r = b3.reshape(1, H)
    bc1r = bc1.reshape(1, 64)
    bc2r = bc2.reshape(1, C_OUT)

    a = sc_scatter(src1, dst1, y1)
    y2 = _tc_mid(a, a, y1, dinv, b1r, W2)
    a = sc_scatter(src1, dst1, y2)
    y3 = _tc_mid(a, a, y2, dinv, b2r, W3)
    a = sc_scatter(src1, dst1, y3)
    out = _tc_final(a, a, y3, dinv, b3r, Wc1, bc1r, Wc2, bc2r)
    return out


# speedup vs baseline: 23.1680x; 1.0669x over previous
import functools

import jax
import jax.numpy as jnp
from jax import lax
from jax.experimental import pallas as pl
from jax.experimental.pallas import tpu as pltpu
from jax.experimental.pallas import tpu_sc as plsc

N = 10000
E = 320000
H = 128
C_OUT = 12

NC = 2
NS = 16
NW = NC * NS
N_PAD = 10240
RPT = N_PAD // NS
CW = 80
EPW = E // NW
NCH = EPW // CW
BR = 2000


def _sc_scatter_body(
    src_hbm, dst_hbm, y_hbm, out_hbm,
    sbuf0, sbuf1, sbuf2, sbuf3, dbuf0, dbuf1, dbuf2, dbuf3,
    rows0, rows1, rows2, rows3, acc,
    isem0, isem1, isem2, isem3, gsem0, gsem1, gsem2, gsem3,
    ssem0, ssem1, ssem2, ssem3,
):
    cid = lax.axis_index("c")
    sid = lax.axis_index("s")
    wid = sid * NC + cid
    base = sid * RPT
    ebase = wid * EPW

    slots = (
        (sbuf0, dbuf0, rows0, isem0, gsem0, ssem0),
        (sbuf1, dbuf1, rows1, isem1, gsem1, ssem1),
        (sbuf2, dbuf2, rows2, isem2, gsem2, ssem2),
        (sbuf3, dbuf3, rows3, isem3, gsem3, ssem3),
    )
    NSL = len(slots)

    zeros16 = jnp.zeros((16,), jnp.float32)

    def zfill(i, _):
        for k in range(H // 16):
            rows0[i, pl.ds(k * 16, 16)] = zeros16
        return 0

    lax.fori_loop(0, CW, zfill, 0)
    for k in range(RPT // CW):
        pltpu.sync_copy(rows0, acc.at[pl.ds(base + k * CW, CW)])
    plsc.subcore_barrier()

    def fetch(j, sl):
        sb, db, _, isem, _, _ = sl
        pltpu.async_copy(src_hbm.at[pl.ds(ebase + j * CW, CW)], sb, isem)
        pltpu.async_copy(dst_hbm.at[pl.ds(ebase + j * CW, CW)], db, isem)

    def wait_fetch(j, sl):
        sb, db, _, isem, _, _ = sl
        pltpu.make_async_copy(src_hbm.at[pl.ds(ebase + j * CW, CW)], sb, isem).wait()
        pltpu.make_async_copy(dst_hbm.at[pl.ds(ebase + j * CW, CW)], db, isem).wait()

    def gather(sl):
        sb, _, rows, _, gsem, _ = sl
        pltpu.async_copy(y_hbm.at[sb], rows, gsem)

    def wait_gather(sl):
        sb, _, rows, _, gsem, _ = sl
        pltpu.make_async_copy(y_hbm.at[sb], rows, gsem).wait()

    def scatter(sl):
        _, db, rows, _, _, ssem = sl
        pltpu.async_copy(rows, acc.at[db], ssem, add=True)

    def wait_scatter(sl):
        _, db, rows, _, _, ssem = sl
        pltpu.make_async_copy(rows, acc.at[db], ssem).wait()

    for sidx in range(NSL):
        fetch(sidx, slots[sidx])
    for sidx in range(NSL):
        wait_fetch(sidx, slots[sidx])
        gather(slots[sidx])
    for sidx in range(NSL):
        wait_gather(slots[sidx])
        scatter(slots[sidx])

    def body(k, _):
        j = NSL * k
        for sidx in range(NSL):
            wait_scatter(slots[sidx])
            fetch(j + sidx, slots[sidx])
        for sidx in range(NSL):
            wait_fetch(j + sidx, slots[sidx])
            gather(slots[sidx])
        for sidx in range(NSL):
            wait_gather(slots[sidx])
            scatter(slots[sidx])
        return 0

    lax.fori_loop(1, NCH // NSL, body, 0)

    wait_scatter(slots[0])
    fetch(NCH - 1, slots[0])
    wait_fetch(NCH - 1, slots[0])
    gather(slots[0])
    wait_gather(slots[0])
    scatter(slots[0])
    for sidx in range(1, NSL):
        wait_scatter(slots[sidx])
    wait_scatter(slots[0])

    plsc.subcore_barrier()
    pltpu.sync_copy(
        acc.at[pl.ds(base, RPT)], out_hbm.at[cid, pl.ds(base, RPT)]
    )


@functools.cache
def _get_sc_scatter():
    mesh = plsc.VectorSubcoreMesh(
        core_axis_name="c", subcore_axis_name="s", num_cores=NC, num_subcores=NS
    )
    return pl.kernel(
        _sc_scatter_body,
        out_type=jax.ShapeDtypeStruct((NC, N_PAD, H), jnp.float32),
        mesh=mesh,
        scratch_types=(
            [pltpu.VMEM((CW,), jnp.int32)] * 8
            + [pltpu.VMEM((CW, H), jnp.float32)] * 4
            + [pltpu.VMEM_SHARED((N_PAD, H), jnp.float32)]
            + [pltpu.SemaphoreType.DMA] * 12
        ),
    )


def _sc_deg_body(dst_hbm, out_hbm, dbuf0, dbuf1, buf, acc, isem0, isem1, ssem0, ssem1):
    cid = lax.axis_index("c")
    sid = lax.axis_index("s")
    wid = sid * NC + cid
    base = sid * RPT
    ebase = wid * EPW

    zeros16 = jnp.zeros((16,), jnp.float32)
    ones16 = jnp.ones((16,), jnp.float32)

    def zfill(i, _):
        buf[pl.ds(i * 16, 16)] = zeros16
        return 0

    lax.fori_loop(0, CW // 16, zfill, 0)
    for k in range(RPT // CW):
        pltpu.sync_copy(buf, acc.at[pl.ds(base + k * CW, CW)])
    plsc.subcore_barrier()

    def ofill(i, _):
        buf[pl.ds(i * 16, 16)] = ones16
        return 0

    lax.fori_loop(0, CW // 16, ofill, 0)

    def fetch(j, db, sem):
        pltpu.async_copy(dst_hbm.at[pl.ds(ebase + j * CW, CW)], db, sem)

    def wait_fetch(j, db, sem):
        pltpu.make_async_copy(dst_hbm.at[pl.ds(ebase + j * CW, CW)], db, sem).wait()

    def scatter(db, sem):
        pltpu.async_copy(buf, acc.at[db], sem, add=True)

    def wait_scatter(db, sem):
        pltpu.make_async_copy(buf, acc.at[db], sem).wait()

    fetch(0, dbuf0, isem0)
    fetch(1, dbuf1, isem1)
    wait_fetch(0, dbuf0, isem0)
    scatter(dbuf0, ssem0)
    wait_fetch(1, dbuf1, isem1)
    scatter(dbuf1, ssem1)

    def body(k, _):
        j0 = 2 * k
        j1 = 2 * k + 1
        wait_scatter(dbuf0, ssem0)
        fetch(j0, dbuf0, isem0)
        wait_scatter(dbuf1, ssem1)
        fetch(j1, dbuf1, isem1)
        wait_fetch(j0, dbuf0, isem0)
        scatter(dbuf0, ssem0)
        wait_fetch(j1, dbuf1, isem1)
        scatter(dbuf1, ssem1)
        return 0

    lax.fori_loop(1, (NCH - 1) // 2, body, 0)

    wait_scatter(dbuf0, ssem0)
    fetch(NCH - 1, dbuf0, isem0)
    wait_fetch(NCH - 1, dbuf0, isem0)
    scatter(dbuf0, ssem0)
    wait_scatter(dbuf1, ssem1)
    wait_scatter(dbuf0, ssem0)

    plsc.subcore_barrier()
    pltpu.sync_copy(
        acc.at[pl.ds(base, RPT)], out_hbm.at[pl.ds(cid * N_PAD + base, RPT)]
    )


@functools.cache
def _get_sc_deg():
    mesh = plsc.VectorSubcoreMesh(
        core_axis_name="c", subcore_axis_name="s", num_cores=NC, num_subcores=NS
    )
    return pl.kernel(
        _sc_deg_body,
        out_type=jax.ShapeDtypeStruct((NC * N_PAD,), jnp.float32),
        mesh=mesh,
        scratch_types=[
            pltpu.VMEM((CW,), jnp.int32),
            pltpu.VMEM((CW,), jnp.int32),
            pltpu.VMEM((CW,), jnp.float32),
            pltpu.VMEM_SHARED((N_PAD,), jnp.float32),
            pltpu.SemaphoreType.DMA,
            pltpu.SemaphoreType.DMA,
            pltpu.SemaphoreType.DMA,
            pltpu.SemaphoreType.DMA,
        ],
    )


def _tc_first_body(x_ref, w_ref, d0_ref, d1_ref, y_ref, dinv_ref):
    deg = d0_ref[...] + d1_ref[...] + 1.0
    dinv = lax.rsqrt(deg)
    xw = jnp.dot(x_ref[...], w_ref[...], preferred_element_type=jnp.float32)
    y_ref[...] = xw * dinv
    dinv_ref[...] = dinv


_tc_first = pl.pallas_call(
    _tc_first_body,
    grid=(N // BR,),
    in_specs=[
        pl.BlockSpec((BR, H), lambda i: (i, 0)),
        pl.BlockSpec((H, H), lambda i: (0, 0)),
        pl.BlockSpec((BR, 1), lambda i: (i, 0)),
        pl.BlockSpec((BR, 1), lambda i: (i, 0)),
    ],
    out_specs=[
        pl.BlockSpec((BR, H), lambda i: (i, 0)),
        pl.BlockSpec((BR, 1), lambda i: (i, 0)),
    ],
    out_shape=[
        jax.ShapeDtypeStruct((N, H), jnp.float32),
        jax.ShapeDtypeStruct((N, 1), jnp.float32),
    ],
)


def _tc_mid_body(a0_ref, a1_ref, y_ref, dinv_ref, b_ref, w_ref, out_ref):
    dinv = dinv_ref[...]
    h = jnp.tanh(dinv * (a0_ref[0] + a1_ref[0] + y_ref[...]) + b_ref[...])
    out_ref[...] = dinv * jnp.dot(
        h, w_ref[...], preferred_element_type=jnp.float32
    )


_tc_mid = pl.pallas_call(
    _tc_mid_body,
    grid=(N // BR,),
    in_specs=[
        pl.BlockSpec((1, BR, H), lambda i: (0, i, 0)),
        pl.BlockSpec((1, BR, H), lambda i: (1, i, 0)),
        pl.BlockSpec((BR, H), lambda i: (i, 0)),
        pl.BlockSpec((BR, 1), lambda i: (i, 0)),
        pl.BlockSpec((1, H), lambda i: (0, 0)),
        pl.BlockSpec((H, H), lambda i: (0, 0)),
    ],
    out_specs=pl.BlockSpec((BR, H), lambda i: (i, 0)),
    out_shape=jax.ShapeDtypeStruct((N, H), jnp.float32),
)


def _tc_final_body(
    a0_ref, a1_ref, y_ref, dinv_ref, b_ref, wc1_ref, bc1_ref, wc2_ref, bc2_ref, out_ref
):
    dinv = dinv_ref[...]
    h = jnp.tanh(dinv * (a0_ref[0] + a1_ref[0] + y_ref[...]) + b_ref[...])
    o = jnp.dot(h, wc1_ref[...], preferred_element_type=jnp.float32) + bc1_ref[...]
    out_ref[...] = (
        jnp.dot(o, wc2_ref[...], preferred_element_type=jnp.float32) + bc2_ref[...]
    )


_tc_final = pl.pallas_call(
    _tc_final_body,
    grid=(N // BR,),
    in_specs=[
        pl.BlockSpec((1, BR, H), lambda i: (0, i, 0)),
        pl.BlockSpec((1, BR, H), lambda i: (1, i, 0)),
        pl.BlockSpec((BR, H), lambda i: (i, 0)),
        pl.BlockSpec((BR, 1), lambda i: (i, 0)),
        pl.BlockSpec((1, H), lambda i: (0, 0)),
        pl.BlockSpec((H, 64), lambda i: (0, 0)),
        pl.BlockSpec((1, 64), lambda i: (0, 0)),
        pl.BlockSpec((64, C_OUT), lambda i: (0, 0)),
        pl.BlockSpec((1, C_OUT), lambda i: (0, 0)),
    ],
    out_specs=pl.BlockSpec((BR, C_OUT), lambda i: (i, 0)),
    out_shape=jax.ShapeDtypeStruct((N, C_OUT), jnp.float32),
)


def kernel(x, edge_index, W1, b1, W2, b2, W3, b3, Wc1, bc1, Wc2, bc2):
    src1 = edge_index[0]
    dst1 = edge_index[1]

    sc_scatter = _get_sc_scatter()
    degp = _get_sc_deg()(dst1)
    d0 = degp[:N].reshape(N, 1)
    d1 = degp[N_PAD : N_PAD + N].reshape(N, 1)

    y1, dinv = _tc_first(x, W1, d0, d1)

    b1r = b1.reshape(1, H)
    b2r = b2.reshape(1, H)
    b3r = b3.reshape(1, H)
    bc1r = bc1.reshape(1, 64)
    bc2r = bc2.reshape(1, C_OUT)

    a = sc_scatter(src1, dst1, y1)
    y2 = _tc_mid(a, a, y1, dinv, b1r, W2)
    a = sc_scatter(src1, dst1, y2)
    y3 = _tc_mid(a, a, y2, dinv, b2r, W3)
    a = sc_scatter(src1, dst1, y3)
    out = _tc_final(a, a, y3, dinv, b3r, Wc1, bc1r, Wc2, bc2r)
    return out


# speedup vs baseline: 23.8992x; 1.0316x over previous
import functools

import jax
import jax.numpy as jnp
from jax import lax
from jax.experimental import pallas as pl
from jax.experimental.pallas import tpu as pltpu
from jax.experimental.pallas import tpu_sc as plsc

N = 10000
E = 320000
H = 128
C_OUT = 12

NC = 2
NS = 16
NW = NC * NS
N_PAD = 10240
RPT = N_PAD // NS
CW = 80
EPW = E // NW
NCH = EPW // CW
BR = 2000


def _sc_scatter_body(
    src_hbm, dst_hbm, y_hbm, out_hbm,
    sbuf0, sbuf1, sbuf2, sbuf3, dbuf0, dbuf1, dbuf2, dbuf3,
    rows0, rows1, rows2, rows3, acc,
    isem0, isem1, isem2, isem3, gsem0, gsem1, gsem2, gsem3,
    ssem0, ssem1, ssem2, ssem3,
):
    cid = lax.axis_index("c")
    sid = lax.axis_index("s")
    wid = sid * NC + cid
    base = sid * RPT
    ebase = wid * EPW

    slots = (
        (sbuf0, dbuf0, rows0, isem0, gsem0, ssem0),
        (sbuf1, dbuf1, rows1, isem1, gsem1, ssem1),
        (sbuf2, dbuf2, rows2, isem2, gsem2, ssem2),
        (sbuf3, dbuf3, rows3, isem3, gsem3, ssem3),
    )
    NSL = len(slots)

    zeros16 = jnp.zeros((16,), jnp.float32)

    def zfill(i, _):
        for k in range(H // 16):
            rows0[i, pl.ds(k * 16, 16)] = zeros16
        return 0

    lax.fori_loop(0, CW, zfill, 0)
    for k in range(RPT // CW):
        pltpu.sync_copy(rows0, acc.at[pl.ds(base + k * CW, CW)])
    plsc.subcore_barrier()

    def fetch(j, sl):
        sb, db, _, isem, _, _ = sl
        pltpu.async_copy(src_hbm.at[pl.ds(ebase + j * CW, CW)], sb, isem)
        pltpu.async_copy(dst_hbm.at[pl.ds(ebase + j * CW, CW)], db, isem)

    def wait_fetch(j, sl):
        sb, db, _, isem, _, _ = sl
        pltpu.make_async_copy(src_hbm.at[pl.ds(ebase + j * CW, CW)], sb, isem).wait()
        pltpu.make_async_copy(dst_hbm.at[pl.ds(ebase + j * CW, CW)], db, isem).wait()

    def gather(sl):
        sb, _, rows, _, gsem, _ = sl
        pltpu.async_copy(y_hbm.at[sb], rows, gsem)

    def wait_gather(sl):
        sb, _, rows, _, gsem, _ = sl
        pltpu.make_async_copy(y_hbm.at[sb], rows, gsem).wait()

    def scatter(sl):
        _, db, rows, _, _, ssem = sl
        pltpu.async_copy(rows, acc.at[db], ssem, add=True)

    def wait_scatter(sl):
        _, db, rows, _, _, ssem = sl
        pltpu.make_async_copy(rows, acc.at[db], ssem).wait()

    for sidx in range(NSL):
        fetch(sidx, slots[sidx])
    for sidx in range(NSL):
        wait_fetch(sidx, slots[sidx])
        gather(slots[sidx])
    for sidx in range(NSL):
        wait_gather(slots[sidx])
        scatter(slots[sidx])

    def body(k, _):
        j = NSL * k
        for sidx in range(NSL):
            wait_scatter(slots[sidx])
            fetch(j + sidx, slots[sidx])
        for sidx in range(NSL):
            wait_fetch(j + sidx, slots[sidx])
            gather(slots[sidx])
        for sidx in range(NSL):
            wait_gather(slots[sidx])
            scatter(slots[sidx])
        return 0

    lax.fori_loop(1, NCH // NSL, body, 0)

    wait_scatter(slots[0])
    fetch(NCH - 1, slots[0])
    wait_fetch(NCH - 1, slots[0])
    gather(slots[0])
    wait_gather(slots[0])
    scatter(slots[0])
    for sidx in range(1, NSL):
        wait_scatter(slots[sidx])
    wait_scatter(slots[0])

    plsc.subcore_barrier()
    pltpu.sync_copy(
        acc.at[pl.ds(base, RPT)], out_hbm.at[cid, pl.ds(base, RPT)]
    )


@functools.cache
def _get_sc_scatter():
    mesh = plsc.VectorSubcoreMesh(
        core_axis_name="c", subcore_axis_name="s", num_cores=NC, num_subcores=NS
    )
    return pl.kernel(
        _sc_scatter_body,
        out_type=jax.ShapeDtypeStruct((NC, N_PAD, H), jnp.float32),
        mesh=mesh,
        scratch_types=(
            [pltpu.VMEM((CW,), jnp.int32)] * 8
            + [pltpu.VMEM((CW, H), jnp.float32)] * 4
            + [pltpu.VMEM_SHARED((N_PAD, H), jnp.float32)]
            + [pltpu.SemaphoreType.DMA] * 12
        ),
    )


def _sc_deg_body(
    dst_hbm, out_hbm, dbuf0, dbuf1, dbuf2, dbuf3, buf, acc,
    isem0, isem1, isem2, isem3, ssem0, ssem1, ssem2, ssem3,
):
    cid = lax.axis_index("c")
    sid = lax.axis_index("s")
    wid = sid * NC + cid
    base = sid * RPT
    ebase = wid * EPW

    slots = (
        (dbuf0, isem0, ssem0),
        (dbuf1, isem1, ssem1),
        (dbuf2, isem2, ssem2),
        (dbuf3, isem3, ssem3),
    )
    NSL = len(slots)

    zeros16 = jnp.zeros((16,), jnp.float32)
    ones16 = jnp.ones((16,), jnp.float32)

    def zfill(i, _):
        buf[pl.ds(i * 16, 16)] = zeros16
        return 0

    lax.fori_loop(0, CW // 16, zfill, 0)
    for k in range(RPT // CW):
        pltpu.sync_copy(buf, acc.at[pl.ds(base + k * CW, CW)])
    plsc.subcore_barrier()

    def ofill(i, _):
        buf[pl.ds(i * 16, 16)] = ones16
        return 0

    lax.fori_loop(0, CW // 16, ofill, 0)

    def fetch(j, sl):
        db, isem, _ = sl
        pltpu.async_copy(dst_hbm.at[pl.ds(ebase + j * CW, CW)], db, isem)

    def wait_fetch(j, sl):
        db, isem, _ = sl
        pltpu.make_async_copy(dst_hbm.at[pl.ds(ebase + j * CW, CW)], db, isem).wait()

    def scatter(sl):
        db, _, ssem = sl
        pltpu.async_copy(buf, acc.at[db], ssem, add=True)

    def wait_scatter(sl):
        db, _, ssem = sl
        pltpu.make_async_copy(buf, acc.at[db], ssem).wait()

    for sidx in range(NSL):
        fetch(sidx, slots[sidx])
    for sidx in range(NSL):
        wait_fetch(sidx, slots[sidx])
        scatter(slots[sidx])

    def body(k, _):
        j = NSL * k
        for sidx in range(NSL):
            wait_scatter(slots[sidx])
            fetch(j + sidx, slots[sidx])
        for sidx in range(NSL):
            wait_fetch(j + sidx, slots[sidx])
            scatter(slots[sidx])
        return 0

    lax.fori_loop(1, NCH // NSL, body, 0)

    wait_scatter(slots[0])
    fetch(NCH - 1, slots[0])
    wait_fetch(NCH - 1, slots[0])
    scatter(slots[0])
    for sidx in range(1, NSL):
        wait_scatter(slots[sidx])
    wait_scatter(slots[0])

    plsc.subcore_barrier()
    pltpu.sync_copy(
        acc.at[pl.ds(base, RPT)], out_hbm.at[pl.ds(cid * N_PAD + base, RPT)]
    )


@functools.cache
def _get_sc_deg():
    mesh = plsc.VectorSubcoreMesh(
        core_axis_name="c", subcore_axis_name="s", num_cores=NC, num_subcores=NS
    )
    return pl.kernel(
        _sc_deg_body,
        out_type=jax.ShapeDtypeStruct((NC * N_PAD,), jnp.float32),
        mesh=mesh,
        scratch_types=(
            [pltpu.VMEM((CW,), jnp.int32)] * 4
            + [pltpu.VMEM((CW,), jnp.float32)]
            + [pltpu.VMEM_SHARED((N_PAD,), jnp.float32)]
            + [pltpu.SemaphoreType.DMA] * 8
        ),
    )


def _tc_first_body(x_ref, w_ref, d0_ref, d1_ref, y_ref, dinv_ref):
    deg = d0_ref[...] + d1_ref[...] + 1.0
    dinv = lax.rsqrt(deg)
    xw = jnp.dot(x_ref[...], w_ref[...], preferred_element_type=jnp.float32)
    y_ref[...] = xw * dinv
    dinv_ref[...] = dinv


_tc_first = pl.pallas_call(
    _tc_first_body,
    grid=(N // BR,),
    in_specs=[
        pl.BlockSpec((BR, H), lambda i: (i, 0)),
        pl.BlockSpec((H, H), lambda i: (0, 0)),
        pl.BlockSpec((BR, 1), lambda i: (i, 0)),
        pl.BlockSpec((BR, 1), lambda i: (i, 0)),
    ],
    out_specs=[
        pl.BlockSpec((BR, H), lambda i: (i, 0)),
        pl.BlockSpec((BR, 1), lambda i: (i, 0)),
    ],
    out_shape=[
        jax.ShapeDtypeStruct((N, H), jnp.float32),
        jax.ShapeDtypeStruct((N, 1), jnp.float32),
    ],
)


def _tc_mid_body(a0_ref, a1_ref, y_ref, dinv_ref, b_ref, w_ref, out_ref):
    dinv = dinv_ref[...]
    h = jnp.tanh(dinv * (a0_ref[0] + a1_ref[0] + y_ref[...]) + b_ref[...])
    out_ref[...] = dinv * jnp.dot(
        h, w_ref[...], preferred_element_type=jnp.float32
    )


_tc_mid = pl.pallas_call(
    _tc_mid_body,
    grid=(N // BR,),
    in_specs=[
        pl.BlockSpec((1, BR, H), lambda i: (0, i, 0)),
        pl.BlockSpec((1, BR, H), lambda i: (1, i, 0)),
        pl.BlockSpec((BR, H), lambda i: (i, 0)),
        pl.BlockSpec((BR, 1), lambda i: (i, 0)),
        pl.BlockSpec((1, H), lambda i: (0, 0)),
        pl.BlockSpec((H, H), lambda i: (0, 0)),
    ],
    out_specs=pl.BlockSpec((BR, H), lambda i: (i, 0)),
    out_shape=jax.ShapeDtypeStruct((N, H), jnp.float32),
)


def _tc_final_body(
    a0_ref, a1_ref, y_ref, dinv_ref, b_ref, wc1_ref, bc1_ref, wc2_ref, bc2_ref, out_ref
):
    dinv = dinv_ref[...]
    h = jnp.tanh(dinv * (a0_ref[0] + a1_ref[0] + y_ref[...]) + b_ref[...])
    o = jnp.dot(h, wc1_ref[...], preferred_element_type=jnp.float32) + bc1_ref[...]
    out_ref[...] = (
        jnp.dot(o, wc2_ref[...], preferred_element_type=jnp.float32) + bc2_ref[...]
    )


_tc_final = pl.pallas_call(
    _tc_final_body,
    grid=(N // BR,),
    in_specs=[
        pl.BlockSpec((1, BR, H), lambda i: (0, i, 0)),
        pl.BlockSpec((1, BR, H), lambda i: (1, i, 0)),
        pl.BlockSpec((BR, H), lambda i: (i, 0)),
        pl.BlockSpec((BR, 1), lambda i: (i, 0)),
        pl.BlockSpec((1, H), lambda i: (0, 0)),
        pl.BlockSpec((H, 64), lambda i: (0, 0)),
        pl.BlockSpec((1, 64), lambda i: (0, 0)),
        pl.BlockSpec((64, C_OUT), lambda i: (0, 0)),
        pl.BlockSpec((1, C_OUT), lambda i: (0, 0)),
    ],
    out_specs=pl.BlockSpec((BR, C_OUT), lambda i: (i, 0)),
    out_shape=jax.ShapeDtypeStruct((N, C_OUT), jnp.float32),
)


def kernel(x, edge_index, W1, b1, W2, b2, W3, b3, Wc1, bc1, Wc2, bc2):
    src1 = edge_index[0]
    dst1 = edge_index[1]

    sc_scatter = _get_sc_scatter()
    degp = _get_sc_deg()(dst1)
    d0 = degp[:N].reshape(N, 1)
    d1 = degp[N_PAD : N_PAD + N].reshape(N, 1)

    y1, dinv = _tc_first(x, W1, d0, d1)

    b1r = b1.reshape(1, H)
    b2r = b2.reshape(1, H)
    b3r = b3.reshape(1, H)
    bc1r = bc1.reshape(1, 64)
    bc2r = bc2.reshape(1, C_OUT)

    a = sc_scatter(src1, dst1, y1)
    y2 = _tc_mid(a, a, y1, dinv, b1r, W2)
    a = sc_scatter(src1, dst1, y2)
    y3 = _tc_mid(a, a, y2, dinv, b2r, W3)
    a = sc_scatter(src1, dst1, y3)
    out = _tc_final(a, a, y3, dinv, b3r, Wc1, bc1r, Wc2, bc2r)
    return out
